# baseline (speedup 1.0000x reference)
import functools

import jax
import jax.numpy as jnp
from jax import lax
from jax.experimental import pallas as pl
from jax.experimental.pallas import tpu as pltpu
from jax.experimental.pallas import tpu_sc as plsc

_BATCH = 16384
_EMB = 32


def _make_sc_gather(batch):
    info = plsc.get_sparse_core_info()
    nw = info.num_cores * info.num_subcores
    per_w = batch // nw
    mesh = plsc.VectorSubcoreMesh(core_axis_name="c", subcore_axis_name="s")

    @functools.partial(
        pl.kernel,
        out_type=(
            jax.ShapeDtypeStruct((batch, 128), jnp.float32),
            jax.ShapeDtypeStruct((batch, 128), jnp.float32),
        ),
        mesh=mesh,
        scratch_types=[
            pltpu.VMEM((per_w // 2,), jnp.int32),
            pltpu.VMEM((per_w // 2,), jnp.int32),
            pltpu.VMEM((per_w // 2,), jnp.int32),
            pltpu.VMEM((per_w // 2,), jnp.int32),
            pltpu.VMEM((per_w // 2, 128), jnp.float32),
            pltpu.VMEM((per_w // 2, 128), jnp.float32),
            pltpu.SemaphoreType.DMA,
            pltpu.SemaphoreType.DMA,
        ],
    )
    def sc_gather(t4_hbm, ia_hbm, ib_hbm, ga_hbm, gb_hbm,
                  ia1_v, ia2_v, ib1_v, ib2_v, ra_v, rb_v, sem_a, sem_b):
        wid = lax.axis_index("s") * info.num_cores + lax.axis_index("c")
        base = wid * per_w
        half = per_w // 2
        pltpu.sync_copy(ia_hbm.at[pl.ds(base, half)], ia1_v)
        pltpu.sync_copy(ib_hbm.at[pl.ds(base, half)], ib1_v)
        pltpu.sync_copy(ia_hbm.at[pl.ds(base + half, half)], ia2_v)
        pltpu.sync_copy(ib_hbm.at[pl.ds(base + half, half)], ib2_v)
        cp = pltpu.async_copy(t4_hbm.at[ia1_v], ra_v, sem_a)
        cp2 = pltpu.async_copy(t4_hbm.at[ib1_v], rb_v, sem_b)
        cp.wait()
        pltpu.sync_copy(ra_v, ga_hbm.at[pl.ds(base, half)])
        cp2.wait()
        pltpu.sync_copy(rb_v, gb_hbm.at[pl.ds(base, half)])
        cp = pltpu.async_copy(t4_hbm.at[ia2_v], ra_v, sem_a)
        cp2 = pltpu.async_copy(t4_hbm.at[ib2_v], rb_v, sem_b)
        cp.wait()
        pltpu.sync_copy(ra_v, ga_hbm.at[pl.ds(base + half, half)])
        cp2.wait()
        pltpu.sync_copy(rb_v, gb_hbm.at[pl.ds(base + half, half)])

    return sc_gather


_sc_gather = _make_sc_gather(_BATCH)


def _dot_t(a, w):
    return lax.dot_general(a, w, (((1,), (1,)), ((), ())),
                           preferred_element_type=jnp.float32)


def _select_quarter(g, oh):
    qlane = lax.broadcasted_iota(jnp.int32, (4, 128), 1) // 32
    qrow = lax.broadcasted_iota(jnp.int32, (4, 128), 0)
    expand = jnp.where(qlane == qrow, 1.0, 0.0)
    m = jnp.dot(oh, expand, preferred_element_type=jnp.float32) * g
    return m[:, 0:32] + m[:, 32:64] + m[:, 64:96] + m[:, 96:128]


def _mlp_body(ga, gb, ma, mb, ft, w1, b1, w2, b2, w3, b3, w4, out):
    xa = _select_quarter(ga[...], ma[...])
    xb = _select_quarter(gb[...], mb[...])
    x = jnp.concatenate([xa, xb, ft[...]], axis=1)
    h = jnp.maximum(_dot_t(x, w1[...]) + b1[...], 0.0)
    h = jnp.maximum(_dot_t(h, w2[...]) + b2[...], 0.0)
    h = jnp.maximum(_dot_t(h, w3[...]) + b3[...], 0.0)
    h = jnp.concatenate([h, jnp.ones((h.shape[0], 1), jnp.float32)], axis=1)
    z = _dot_t(h, w4[...])
    out[...] = jax.nn.sigmoid(z)


def _mlp(ga, gb, ma, mb, ft, W1, b1, W2, b2, W3, b3, W4, blk=2048):
    batch = ga.shape[0]
    grid = (batch // blk,)
    full = lambda a: pl.BlockSpec(a.shape, lambda i: (0,) * a.ndim)
    row = lambda a: pl.BlockSpec((blk, a.shape[1]), lambda i: (i, 0))
    return pl.pallas_call(
        _mlp_body,
        grid=grid,
        in_specs=[
            row(ga), row(gb), row(ma), row(mb), row(ft),
            full(W1), full(b1), full(W2), full(b2),
            full(W3), full(b3), full(W4),
        ],
        out_specs=pl.BlockSpec((blk, 1), lambda i: (i, 0)),
        out_shape=jax.ShapeDtypeStruct((batch, 1), jnp.float32),
    )(ga, gb, ma, mb, ft, W1, b1, W2, b2, W3, b3, W4)


def kernel(airport_a, airport_b, features, table,
           W1, b1, W2, b2, W3, b3, W4, b4):
    ia = airport_a.astype(jnp.int32)
    ib = airport_b.astype(jnp.int32)
    t4 = table.reshape(25000, 128)
    ga, gb = _sc_gather(t4, ia // 4, ib // 4)
    ma = jax.nn.one_hot(ia % 4, 4, dtype=jnp.float32)
    mb = jax.nn.one_hot(ib % 4, 4, dtype=jnp.float32)
    w4e = jnp.concatenate([W4, b4.reshape(1, 1)], axis=1)
    out = _mlp(ga, gb, ma, mb, features,
               W1, b1.reshape(1, -1), W2, b2.reshape(1, -1),
               W3, b3.reshape(1, -1), w4e)
    return out[:, 0]


# speedup vs baseline: 1.3613x; 1.3613x over previous
import functools

import jax
import jax.numpy as jnp
from jax import lax
from jax.experimental import pallas as pl
from jax.experimental.pallas import tpu as pltpu
from jax.experimental.pallas import tpu_sc as plsc

_BATCH = 16384
_EMB = 32


def _make_sc_gather(batch, emb_dim):
    info = plsc.get_sparse_core_info()
    nw = info.num_cores * info.num_subcores
    per_w = batch // nw
    mesh = plsc.VectorSubcoreMesh(core_axis_name="c", subcore_axis_name="s")

    @functools.partial(
        pl.kernel,
        out_type=jax.ShapeDtypeStruct((batch, 128), jnp.float32),
        mesh=mesh,
        compiler_params=pltpu.CompilerParams(use_tc_tiling_on_sc=False),
        scratch_types=[
            pltpu.VMEM((per_w,), jnp.int32),
            pltpu.VMEM((per_w,), jnp.int32),
            pltpu.VMEM((per_w, emb_dim), jnp.float32),
            pltpu.VMEM((per_w, emb_dim), jnp.float32),
            pltpu.SemaphoreType.DMA,
            pltpu.SemaphoreType.DMA,
        ],
    )
    def sc_gather(table_hbm, ia_hbm, ib_hbm, comb_hbm,
                  ia_v, ib_v, ra_v, rb_v, sem_a, sem_b):
        wid = lax.axis_index("s") * info.num_cores + lax.axis_index("c")
        base = wid * per_w
        pltpu.sync_copy(ia_hbm.at[pl.ds(base, per_w)], ia_v)
        pltpu.sync_copy(ib_hbm.at[pl.ds(base, per_w)], ib_v)
        cp_a = pltpu.async_copy(table_hbm.at[ia_v], ra_v, sem_a)
        cp_b = pltpu.async_copy(table_hbm.at[ib_v], rb_v, sem_b)
        cp_a.wait()
        pltpu.sync_copy(ra_v, comb_hbm.at[pl.ds(base, per_w), pl.ds(0, emb_dim)])
        cp_b.wait()
        pltpu.sync_copy(rb_v, comb_hbm.at[pl.ds(base, per_w),
                                          pl.ds(emb_dim, emb_dim)])

    return sc_gather


_sc_gather = _make_sc_gather(_BATCH, _EMB)


def _dot_t(a, w):
    return lax.dot_general(a, w, (((1,), (1,)), ((), ())),
                           preferred_element_type=jnp.float32)


def _mlp_body(comb, ft, w1, b1, w2, b2, w3, b3, w4, out):
    x = jnp.concatenate([comb[:, 0:64], ft[...]], axis=1)
    h = jnp.maximum(_dot_t(x, w1[...]) + b1[...], 0.0)
    h = jnp.maximum(_dot_t(h, w2[...]) + b2[...], 0.0)
    h = jnp.maximum(_dot_t(h, w3[...]) + b3[...], 0.0)
    h = jnp.concatenate([h, jnp.ones((h.shape[0], 1), jnp.float32)], axis=1)
    out[...] = jax.nn.sigmoid(_dot_t(h, w4[...]))


def _mlp(comb, ft, W1, b1, W2, b2, W3, b3, W4e, blk=2048):
    batch = comb.shape[0]
    grid = (batch // blk,)
    full = lambda a: pl.BlockSpec(a.shape, lambda i: (0,) * a.ndim)
    row = lambda a: pl.BlockSpec((blk, a.shape[1]), lambda i: (i, 0))
    return pl.pallas_call(
        _mlp_body,
        grid=grid,
        in_specs=[
            row(comb), row(ft),
            full(W1), full(b1), full(W2), full(b2),
            full(W3), full(b3), full(W4e),
        ],
        out_specs=pl.BlockSpec((blk, 1), lambda i: (i, 0)),
        out_shape=jax.ShapeDtypeStruct((batch, 1), jnp.float32),
    )(comb, ft, W1, b1, W2, b2, W3, b3, W4e)


def kernel(airport_a, airport_b, features, table,
           W1, b1, W2, b2, W3, b3, W4, b4):
    ia = airport_a.astype(jnp.int32)
    ib = airport_b.astype(jnp.int32)
    comb = _sc_gather(table, ia, ib)
    w4e = jnp.concatenate([W4, b4.reshape(1, 1)], axis=1)
    out = _mlp(comb, features,
               W1, b1.reshape(1, -1), W2, b2.reshape(1, -1),
               W3, b3.reshape(1, -1), w4e)
    return out[:, 0]
